# Initial kernel scaffold; baseline (speedup 1.0000x reference)
import functools

import jax
import jax.numpy as jnp
from jax import lax
from jax.experimental import pallas as pl
from jax.experimental.pallas import tpu as pltpu
from jax.experimental.pallas import tpu_sc as plsc

N_NODES = 10000
N_EDGES = 320000
D = 128
L = 16
NC = 2
NS = 16
NW = NC * NS
E_TILE = 10240
E_PAD = NW * E_TILE
CH = 128
NCHUNK = E_TILE // CH
ZR = N_NODES // NS
ZCH = 125

_mesh = plsc.VectorSubcoreMesh(core_axis_name="c", subcore_axis_name="s")


@functools.partial(
    pl.kernel,
    out_type=jax.ShapeDtypeStruct((NC, N_NODES, D), jnp.float32),
    mesh=_mesh,
    scratch_types=[
        pltpu.VMEM((E_TILE,), jnp.int32),
        pltpu.VMEM((NCHUNK, CH), jnp.int32),
        pltpu.VMEM((E_TILE,), jnp.float32),
        pltpu.VMEM((CH, D), jnp.float32),
        pltpu.VMEM_SHARED((N_NODES, D), jnp.float32),
        pltpu.SemaphoreType.DMA,
    ],
)
def _accum(x_hbm, src_hbm, dst_hbm, w_hbm, part_hbm,
           src_v, dst_v, w_v, rows_v, acc_sh, sem):
    cid = lax.axis_index("c")
    sid = lax.axis_index("s")
    wid = cid * NS + sid

    def _zero_row(i, carry):
        for f in range(D // L):
            rows_v[i, pl.ds(f * L, L)] = jnp.zeros((L,), jnp.float32)
        return carry

    lax.fori_loop(0, ZCH, _zero_row, 0)
    for k in range(ZR // ZCH):
        off = sid * ZR + k * ZCH
        pltpu.sync_copy(rows_v.at[pl.ds(0, ZCH)], acc_sh.at[pl.ds(off, ZCH)])
    plsc.subcore_barrier()

    pltpu.sync_copy(src_hbm.at[wid], src_v)
    pltpu.sync_copy(dst_hbm.at[wid], dst_v)
    pltpu.sync_copy(w_hbm.at[wid], w_v)

    def _chunk(j, carry):
        idx = src_v.at[pl.ds(j * CH, CH)]
        pltpu.async_copy(x_hbm.at[idx], rows_v, sem).wait()

        def _scale(e, c2):
            w = w_v[j * CH + e]
            for f in range(D // L):
                sl = pl.ds(f * L, L)
                rows_v[e, sl] = rows_v[e, sl] * w
            return c2

        lax.fori_loop(0, CH, _scale, 0)
        pltpu.sync_copy(rows_v, acc_sh.at[dst_v.at[j]], add=True)
        return carry

    lax.fori_loop(0, NCHUNK, _chunk, 0)
    plsc.subcore_barrier()

    for k in range(ZR // ZCH):
        off = sid * ZR + k * ZCH
        pltpu.sync_copy(acc_sh.at[pl.ds(off, ZCH)], rows_v.at[pl.ds(0, ZCH)])
        pltpu.sync_copy(rows_v.at[pl.ds(0, ZCH)],
                        part_hbm.at[cid, pl.ds(off, ZCH)])


@functools.partial(
    pl.kernel,
    out_type=jax.ShapeDtypeStruct((N_NODES, D), jnp.float32),
    mesh=_mesh,
    scratch_types=[
        pltpu.VMEM((ZCH, D), jnp.float32),
        pltpu.VMEM((ZCH, D), jnp.float32),
    ],
)
def _combine(part_hbm, out_hbm, a_v, b_v):
    cid = lax.axis_index("c")
    sid = lax.axis_index("s")
    wid = cid * NS + sid
    n_out_chunks = N_NODES // ZCH

    for k in range(3):
        c = wid + NW * k

        @pl.when(c < n_out_chunks)
        def _():
            off = c * ZCH
            pltpu.sync_copy(part_hbm.at[0, pl.ds(off, ZCH)], a_v)
            pltpu.sync_copy(part_hbm.at[1, pl.ds(off, ZCH)], b_v)

            def _add(i, carry):
                for f in range(D // L):
                    sl = pl.ds(f * L, L)
                    a_v[i, sl] = a_v[i, sl] + b_v[i, sl]
                return carry

            lax.fori_loop(0, ZCH, _add, 0)
            pltpu.sync_copy(a_v, out_hbm.at[pl.ds(off, ZCH)])


def kernel(x, edge_index, edge_weight):
    dst = edge_index[0]
    src = edge_index[1]
    pad = E_PAD - N_EDGES
    src_p = jnp.concatenate(
        [src.astype(jnp.int32), jnp.zeros((pad,), jnp.int32)]
    ).reshape(NW, E_TILE)
    dst_p = jnp.concatenate(
        [dst.astype(jnp.int32), jnp.zeros((pad,), jnp.int32)]
    ).reshape(NW, NCHUNK, CH)
    w_p = jnp.concatenate(
        [edge_weight, jnp.zeros((pad,), jnp.float32)]
    ).reshape(NW, E_TILE)
    part = _accum(x, src_p, dst_p, w_p)
    return _combine(part)


# speedup vs baseline: 2.8197x; 2.8197x over previous
import functools

import jax
import jax.numpy as jnp
from jax import lax
from jax.experimental import pallas as pl
from jax.experimental.pallas import tpu as pltpu
from jax.experimental.pallas import tpu_sc as plsc

N_NODES = 10000
N_EDGES = 320000
D = 128
L = 16
NC = 2
NS = 16
NW = NC * NS
E_TILE = 10240
E_PAD = NW * E_TILE
CH = 128
NCHUNK = E_TILE // CH
ZCH = 80
NZCH = N_NODES // ZCH

_mesh = plsc.VectorSubcoreMesh(core_axis_name="c", subcore_axis_name="s")


@functools.partial(
    pl.kernel,
    out_type=jax.ShapeDtypeStruct((NC, N_NODES, D), jnp.float32),
    mesh=_mesh,
    scratch_types=[
        pltpu.VMEM((E_TILE,), jnp.int32),
        pltpu.VMEM((NCHUNK, CH), jnp.int32),
        pltpu.VMEM((E_TILE,), jnp.float32),
        pltpu.VMEM((CH, D), jnp.float32),
        pltpu.VMEM_SHARED((N_NODES, D), jnp.float32),
        pltpu.SemaphoreType.DMA,
    ],
)
def _accum(x_hbm, src_hbm, dst_hbm, w_hbm, part_hbm,
           src_v, dst_v, w_v, rows_v, acc_sh, sem):
    cid = lax.axis_index("c")
    sid = lax.axis_index("s")
    wid = cid * NS + sid

    def _zero_row(i, carry):
        for f in range(D // L):
            rows_v[i, pl.ds(f * L, L)] = jnp.zeros((L,), jnp.float32)
        return carry

    lax.fori_loop(0, ZCH, _zero_row, 0)
    for k in range(NZCH // NS + 1):
        c = sid + NS * k

        @pl.when(c < NZCH)
        def _():
            pltpu.sync_copy(rows_v.at[pl.ds(0, ZCH)],
                            acc_sh.at[pl.ds(c * ZCH, ZCH)])

    plsc.subcore_barrier()

    pltpu.sync_copy(src_hbm.at[wid], src_v)
    pltpu.sync_copy(dst_hbm.at[wid], dst_v)
    pltpu.sync_copy(w_hbm.at[wid], w_v)

    def _chunk(j, carry):
        idx = src_v.at[pl.ds(j * CH, CH)]
        pltpu.async_copy(x_hbm.at[idx], rows_v, sem).wait()

        def _scale(g, c2):
            wvec = w_v[pl.ds(j * CH + g * L, L)]
            for e16 in range(L):
                w = wvec[e16]
                e = g * L + e16
                for f in range(D // L):
                    sl = pl.ds(f * L, L)
                    rows_v[e, sl] = rows_v[e, sl] * w
            return c2

        lax.fori_loop(0, CH // L, _scale, 0)
        pltpu.sync_copy(rows_v, acc_sh.at[dst_v.at[j]], add=True)
        return carry

    lax.fori_loop(0, NCHUNK, _chunk, 0)
    plsc.subcore_barrier()

    for k in range(NZCH // NS + 1):
        c = sid + NS * k

        @pl.when(c < NZCH)
        def _():
            off = c * ZCH
            pltpu.sync_copy(acc_sh.at[pl.ds(off, ZCH)],
                            rows_v.at[pl.ds(0, ZCH)])
            pltpu.sync_copy(rows_v.at[pl.ds(0, ZCH)],
                            part_hbm.at[cid, pl.ds(off, ZCH)])


@functools.partial(
    pl.kernel,
    out_type=jax.ShapeDtypeStruct((N_NODES, D), jnp.float32),
    mesh=_mesh,
    scratch_types=[
        pltpu.VMEM((ZCH, D), jnp.float32),
        pltpu.VMEM((ZCH, D), jnp.float32),
    ],
)
def _combine(part_hbm, out_hbm, a_v, b_v):
    cid = lax.axis_index("c")
    sid = lax.axis_index("s")
    wid = cid * NS + sid

    for k in range(NZCH // NW + 1):
        c = wid + NW * k

        @pl.when(c < NZCH)
        def _():
            off = c * ZCH
            pltpu.sync_copy(part_hbm.at[0, pl.ds(off, ZCH)], a_v)
            pltpu.sync_copy(part_hbm.at[1, pl.ds(off, ZCH)], b_v)

            def _add(i, carry):
                for f in range(D // L):
                    sl = pl.ds(f * L, L)
                    a_v[i, sl] = a_v[i, sl] + b_v[i, sl]
                return carry

            lax.fori_loop(0, ZCH, _add, 0)
            pltpu.sync_copy(a_v, out_hbm.at[pl.ds(off, ZCH)])


def kernel(x, edge_index, edge_weight):
    dst = edge_index[0]
    src = edge_index[1]
    pad = E_PAD - N_EDGES
    src_p = jnp.concatenate(
        [src.astype(jnp.int32), jnp.zeros((pad,), jnp.int32)]
    ).reshape(NW, E_TILE)
    dst_p = jnp.concatenate(
        [dst.astype(jnp.int32), jnp.zeros((pad,), jnp.int32)]
    ).reshape(NW, NCHUNK, CH)
    w_p = jnp.concatenate(
        [edge_weight, jnp.zeros((pad,), jnp.float32)]
    ).reshape(NW, E_TILE)
    part = _accum(x, src_p, dst_p, w_p)
    return _combine(part)


# speedup vs baseline: 3.0880x; 1.0951x over previous
import functools

import jax
import jax.numpy as jnp
from jax import lax
from jax.experimental import pallas as pl
from jax.experimental.pallas import tpu as pltpu
from jax.experimental.pallas import tpu_sc as plsc

N_NODES = 10000
N_EDGES = 320000
D = 128
L = 16
NC = 2
NS = 16
NW = NC * NS
CH = 64
BLK = 32
NBLK = 5
E_BLK = BLK * CH
E_TILE = NBLK * E_BLK
E_PAD = NW * E_TILE
NCHUNK = E_TILE // CH
ZCH = 40
NZCH = N_NODES // ZCH

_mesh = plsc.VectorSubcoreMesh(core_axis_name="c", subcore_axis_name="s")


@functools.partial(
    pl.kernel,
    out_type=jax.ShapeDtypeStruct((NC, N_NODES, D), jnp.float32),
    mesh=_mesh,
    scratch_types=[
        pltpu.VMEM((2, E_BLK), jnp.int32),
        pltpu.VMEM((2, BLK, CH), jnp.int32),
        pltpu.VMEM((2, E_BLK), jnp.float32),
        pltpu.VMEM((CH, D), jnp.float32),
        pltpu.VMEM((CH, D), jnp.float32),
        pltpu.VMEM_SHARED((N_NODES, D), jnp.float32),
        pltpu.SemaphoreType.DMA,
        pltpu.SemaphoreType.DMA,
        pltpu.SemaphoreType.DMA,
        pltpu.SemaphoreType.DMA,
        pltpu.SemaphoreType.DMA,
        pltpu.SemaphoreType.DMA,
    ],
)
def _accum(x_hbm, src_hbm, dst_hbm, w_hbm, part_hbm,
           srcb, dstb, wb, r0, r1, acc_sh, b0, b1, g0, g1, s0, s1):
    cid = lax.axis_index("c")
    sid = lax.axis_index("s")
    wid = cid * NS + sid
    bufs = ((r0, g0, s0), (r1, g1, s1))
    bsems = (b0, b1)

    def _zero_row(i, carry):
        for f in range(D // L):
            r0[i, pl.ds(f * L, L)] = jnp.zeros((L,), jnp.float32)
        return carry

    lax.fori_loop(0, ZCH, _zero_row, 0)
    for k in range(NZCH // NS + 1):
        c = sid + NS * k

        @pl.when(c < NZCH)
        def _():
            pltpu.sync_copy(r0.at[pl.ds(0, ZCH)],
                            acc_sh.at[pl.ds(c * ZCH, ZCH)])

    plsc.subcore_barrier()

    def _fire_block(blk):
        slot = blk % 2
        sem = bsems[slot]
        off = wid * E_TILE + blk * E_BLK
        pltpu.async_copy(src_hbm.at[pl.ds(off, E_BLK)], srcb.at[slot], sem)
        pltpu.async_copy(dst_hbm.at[wid * NBLK + blk], dstb.at[slot], sem)
        pltpu.async_copy(w_hbm.at[pl.ds(off, E_BLK)], wb.at[slot], sem)

    def _wait_block(blk):
        slot = blk % 2
        sem = bsems[slot]
        off = wid * E_TILE + blk * E_BLK
        pltpu.make_async_copy(src_hbm.at[pl.ds(off, E_BLK)],
                              srcb.at[slot], sem).wait()
        pltpu.make_async_copy(dst_hbm.at[wid * NBLK + blk],
                              dstb.at[slot], sem).wait()
        pltpu.make_async_copy(w_hbm.at[pl.ds(off, E_BLK)],
                              wb.at[slot], sem).wait()

    def _gather_src(slot, c):
        return x_hbm.at[srcb.at[slot, pl.ds(c * CH, CH)]]

    def _scale_chunk(rows, slot, c):
        def _scale(g, c2):
            wvec = wb[slot, pl.ds(c * CH + g * L, L)]
            for e16 in range(L):
                w = wvec[e16]
                e = g * L + e16
                for f in range(D // L):
                    sl = pl.ds(f * L, L)
                    rows[e, sl] = rows[e, sl] * w
            return c2

        lax.fori_loop(0, CH // L, _scale, 0)

    _fire_block(0)
    _fire_block(1)
    _wait_block(0)
    for b, (rows, gsem, _) in enumerate(bufs):
        pltpu.async_copy(_gather_src(0, b), rows, gsem)

    for blk in range(NBLK):
        slot = blk % 2
        if blk + 1 < NBLK:
            _wait_block(blk + 1)

        def _round(p, carry):
            for b, (rows, gsem, ssem) in enumerate(bufs):
                c = 2 * p + b
                pltpu.make_async_copy(_gather_src(slot, c), rows,
                                      gsem).wait()
                _scale_chunk(rows, slot, c)
                pltpu.async_copy(rows, acc_sh.at[dstb.at[slot, c]], ssem,
                                 add=True)
            for b, (rows, gsem, ssem) in enumerate(bufs):
                c = 2 * p + b
                pltpu.make_async_copy(rows, acc_sh.at[dstb.at[slot, c]],
                                      ssem).wait()
                pltpu.async_copy(_gather_src(slot, c + 2), rows, gsem)
            return carry

        lax.fori_loop(0, BLK // 2 - 1, _round, 0)

        for b, (rows, gsem, ssem) in enumerate(bufs):
            c = BLK - 2 + b
            pltpu.make_async_copy(_gather_src(slot, c), rows, gsem).wait()
            _scale_chunk(rows, slot, c)
            pltpu.async_copy(rows, acc_sh.at[dstb.at[slot, c]], ssem,
                             add=True)
        for b, (rows, gsem, ssem) in enumerate(bufs):
            c = BLK - 2 + b
            pltpu.make_async_copy(rows, acc_sh.at[dstb.at[slot, c]],
                                  ssem).wait()
            if blk + 1 < NBLK:
                pltpu.async_copy(_gather_src((blk + 1) % 2, b), rows, gsem)

        if blk + 2 < NBLK:
            _fire_block(blk + 2)

    plsc.subcore_barrier()

    for k in range(NZCH // NS + 1):
        c = sid + NS * k

        @pl.when(c < NZCH)
        def _():
            off = c * ZCH
            pltpu.sync_copy(acc_sh.at[pl.ds(off, ZCH)], r0.at[pl.ds(0, ZCH)])
            pltpu.sync_copy(r0.at[pl.ds(0, ZCH)],
                            part_hbm.at[cid, pl.ds(off, ZCH)])


@functools.partial(
    pl.kernel,
    out_type=jax.ShapeDtypeStruct((N_NODES, D), jnp.float32),
    mesh=_mesh,
    scratch_types=[
        pltpu.VMEM((ZCH, D), jnp.float32),
        pltpu.VMEM((ZCH, D), jnp.float32),
    ],
)
def _combine(part_hbm, out_hbm, a_v, b_v):
    cid = lax.axis_index("c")
    sid = lax.axis_index("s")
    wid = cid * NS + sid

    for k in range(NZCH // NW + 1):
        c = wid + NW * k

        @pl.when(c < NZCH)
        def _():
            off = c * ZCH
            pltpu.sync_copy(part_hbm.at[0, pl.ds(off, ZCH)], a_v)
            pltpu.sync_copy(part_hbm.at[1, pl.ds(off, ZCH)], b_v)

            def _add(i, carry):
                for f in range(D // L):
                    sl = pl.ds(f * L, L)
                    a_v[i, sl] = a_v[i, sl] + b_v[i, sl]
                return carry

            lax.fori_loop(0, ZCH, _add, 0)
            pltpu.sync_copy(a_v, out_hbm.at[pl.ds(off, ZCH)])


def kernel(x, edge_index, edge_weight):
    dst = edge_index[0]
    src = edge_index[1]
    pad = E_PAD - N_EDGES
    src_p = jnp.concatenate([src.astype(jnp.int32),
                             jnp.zeros((pad,), jnp.int32)])
    dst_p = jnp.concatenate([dst.astype(jnp.int32),
                             jnp.zeros((pad,), jnp.int32)]).reshape(
                                 NW * NBLK, BLK, CH)
    w_p = jnp.concatenate([edge_weight, jnp.zeros((pad,), jnp.float32)])
    part = _accum(x, src_p, dst_p, w_p)
    return _combine(part)


# speedup vs baseline: 8.2996x; 2.6877x over previous
import functools

import jax
import jax.numpy as jnp
from jax import lax
from jax.experimental import pallas as pl
from jax.experimental.pallas import tpu as pltpu
from jax.experimental.pallas import tpu_sc as plsc

N_NODES = 10000
N_EDGES = 320000
D = 128
DP = D // 2
L = 16
NC = 2
NS = 16
NW = NC * NS
CH = 64
BLK = 40
NBLK = 4
E_BLK = BLK * CH
E_TILE = NBLK * E_BLK
E_PAD = NW * E_TILE
NCHUNK = E_TILE // CH
ZCH = 40
NZCH = N_NODES // ZCH
HI_MASK = -65536

_mesh = plsc.VectorSubcoreMesh(core_axis_name="c", subcore_axis_name="s")


@functools.partial(
    pl.kernel,
    out_type=jax.ShapeDtypeStruct((NC, N_NODES, D), jnp.float32),
    mesh=_mesh,
    compiler_params=pltpu.CompilerParams(use_tc_tiling_on_sc=False),
    scratch_types=[
        pltpu.VMEM((2, E_BLK), jnp.int32),
        pltpu.VMEM((2, BLK, CH), jnp.int32),
        pltpu.VMEM((2, E_BLK), jnp.float32),
        pltpu.VMEM((CH, D), jnp.float32),
        pltpu.VMEM((CH, D), jnp.float32),
        pltpu.VMEM_SHARED((N_NODES, D), jnp.float32),
        pltpu.SemaphoreType.DMA,
        pltpu.SemaphoreType.DMA,
        pltpu.SemaphoreType.DMA,
        pltpu.SemaphoreType.DMA,
        pltpu.SemaphoreType.DMA,
    ],
)
def _accum(x_hbm, src_hbm, dst_hbm, w_hbm, part_hbm,
           srcb, dstb, wb, rfa, rfb, acc_sh,
           bsem, g0, g1, s0, s1):
    cid = lax.axis_index("c")
    sid = lax.axis_index("s")
    wid = cid * NS + sid
    bufs = ((rfa, g0, s0), (rfb, g1, s1))

    def _zero_row(i, carry):
        for f in range(D // L):
            rfa[i, pl.ds(f * L, L)] = jnp.zeros((L,), jnp.float32)
        return carry

    lax.fori_loop(0, ZCH, _zero_row, 0)
    for k in range(NZCH // NS + 1):
        c = sid + NS * k

        @pl.when(c < NZCH)
        def _():
            pltpu.sync_copy(rfa.at[pl.ds(0, ZCH)],
                            acc_sh.at[pl.ds(c * ZCH, ZCH)])

    plsc.subcore_barrier()

    def _fire_block(blk, slot):
        off = wid * E_TILE + blk * E_BLK
        pltpu.async_copy(src_hbm.at[pl.ds(off, E_BLK)], srcb.at[slot], bsem)
        pltpu.async_copy(dst_hbm.at[wid * NBLK + blk], dstb.at[slot], bsem)
        pltpu.async_copy(w_hbm.at[pl.ds(off, E_BLK)], wb.at[slot], bsem)

    def _wait_block(blk, slot):
        off = wid * E_TILE + blk * E_BLK
        pltpu.make_async_copy(src_hbm.at[pl.ds(off, E_BLK)],
                              srcb.at[slot], bsem).wait()
        pltpu.make_async_copy(dst_hbm.at[wid * NBLK + blk],
                              dstb.at[slot], bsem).wait()
        pltpu.make_async_copy(w_hbm.at[pl.ds(off, E_BLK)],
                              wb.at[slot], bsem).wait()

    def _gather_src(slot, c):
        return x_hbm.at[srcb.at[slot, pl.ds(c * CH, CH)]]

    def _scale_chunk(slot, c, rows):
        def _cs(g, carry):
            wvec = wb[slot, pl.ds(c * CH + g * L, L)]
            for e16 in range(L):
                w = wvec[e16]
                e = g * L + e16
                for f in range(D // L):
                    sl = pl.ds(f * L, L)
                    rows[e, sl] = rows[e, sl] * w
            return carry

        lax.fori_loop(0, CH // L, _cs, 0)

    _fire_block(0, 0)
    _wait_block(0, 0)
    for b, (rows, gsem, ssem) in enumerate(bufs):
        pltpu.async_copy(_gather_src(0, b), rows, gsem)

    def _pair(p, carry):
        j0 = 2 * p
        blk = j0 // BLK
        slot = lax.rem(blk, 2)
        c0 = j0 - blk * BLK

        @pl.when((c0 == BLK - 2) & (blk + 1 < NBLK))
        def _():
            _wait_block(blk + 1, 1 - slot)

        for b, (rows, gsem, ssem) in enumerate(bufs):
            c = c0 + b
            pltpu.make_async_copy(_gather_src(slot, c), rows, gsem).wait()
            _scale_chunk(slot, c, rows)
            pltpu.async_copy(rows, acc_sh.at[dstb.at[slot, c]], ssem,
                             add=True)

        j2 = j0 + 2
        blk2 = j2 // BLK
        slot2 = lax.rem(blk2, 2)
        c2 = j2 - blk2 * BLK

        for b, (rows, gsem, ssem) in enumerate(bufs):
            c = c0 + b
            pltpu.make_async_copy(rows, acc_sh.at[dstb.at[slot, c]],
                                  ssem).wait()

            @pl.when(j2 < NCHUNK)
            def _():
                pltpu.async_copy(_gather_src(slot2, c2 + b), rows, gsem)

        @pl.when((c0 == 0) & (blk + 1 < NBLK))
        def _():
            _fire_block(blk + 1, 1 - slot)

        return carry

    lax.fori_loop(0, NCHUNK // 2, _pair, 0)

    plsc.subcore_barrier()

    for k in range(NZCH // NS + 1):
        c = sid + NS * k

        @pl.when(c < NZCH)
        def _():
            off = c * ZCH
            pltpu.sync_copy(acc_sh.at[pl.ds(off, ZCH)],
                            rfa.at[pl.ds(0, ZCH)])
            pltpu.sync_copy(rfa.at[pl.ds(0, ZCH)],
                            part_hbm.at[cid, pl.ds(off, ZCH)])


@functools.partial(
    pl.kernel,
    out_type=jax.ShapeDtypeStruct((N_NODES, D), jnp.float32),
    mesh=_mesh,
    scratch_types=[
        pltpu.VMEM((ZCH, D), jnp.float32),
        pltpu.VMEM((ZCH, D), jnp.float32),
        pltpu.VMEM((ZCH, D), jnp.float32),
    ],
)
def _combine(part_hbm, out_hbm, a_v, b_v, c_v):
    cid = lax.axis_index("c")
    sid = lax.axis_index("s")
    wid = cid * NS + sid

    for k in range(NZCH // NW + 1):
        c = wid + NW * k

        @pl.when(c < NZCH)
        def _():
            off = c * ZCH
            pltpu.sync_copy(part_hbm.at[0, pl.ds(off, ZCH)], a_v)
            pltpu.sync_copy(part_hbm.at[1, pl.ds(off, ZCH)], b_v)

            def _add(i, carry):
                for q in range(D // 32):
                    slo = pl.ds(32 * q, L)
                    shi = pl.ds(32 * q + 16, L)
                    c_v[i, slo] = a_v[i, slo] + b_v[i, slo]
                    c_v[i, shi] = a_v[i, shi] + b_v[i, shi]
                return carry

            lax.fori_loop(0, ZCH, _add, 0)
            pltpu.sync_copy(c_v, out_hbm.at[pl.ds(off, ZCH)])


def kernel(x, edge_index, edge_weight):
    dst = edge_index[0]
    src = edge_index[1]
    pad = E_PAD - N_EDGES
    pad_idx = jnp.arange(pad, dtype=jnp.int32) % N_NODES
    src_p = jnp.concatenate([src.astype(jnp.int32), pad_idx])
    dst_p = jnp.concatenate([dst.astype(jnp.int32), pad_idx]).reshape(
        NW * NBLK, BLK, CH)
    w_p = jnp.concatenate([edge_weight, jnp.zeros((pad,), jnp.float32)])
    part = _accum(x, src_p, dst_p, w_p)
    return _combine(part)


# speedup vs baseline: 9.5115x; 1.1460x over previous
import functools

import jax
import jax.numpy as jnp
from jax import lax
from jax.experimental import pallas as pl
from jax.experimental.pallas import tpu as pltpu
from jax.experimental.pallas import tpu_sc as plsc

N_NODES = 10000
N_EDGES = 320000
D = 128
DP = D // 2
L = 16
NC = 2
NS = 16
NW = NC * NS
CH = 64
BLK = 40
NBLK = 4
E_BLK = BLK * CH
E_TILE = NBLK * E_BLK
E_PAD = NW * E_TILE
NCHUNK = E_TILE // CH
ZCH = 40
NZCH = N_NODES // ZCH
HI_MASK = -65536

_mesh = plsc.VectorSubcoreMesh(core_axis_name="c", subcore_axis_name="s")


@functools.partial(
    pl.kernel,
    out_type=jax.ShapeDtypeStruct((NC, N_NODES, D), jnp.float32),
    mesh=_mesh,
    compiler_params=pltpu.CompilerParams(use_tc_tiling_on_sc=False),
    scratch_types=[
        pltpu.VMEM((2, E_BLK), jnp.int32),
        pltpu.VMEM((2, BLK, CH), jnp.int32),
        pltpu.VMEM((2, E_BLK), jnp.float32),
        pltpu.VMEM((CH, D), jnp.float32),
        pltpu.VMEM((CH, D), jnp.float32),
        pltpu.VMEM((CH, D), jnp.float32),
        pltpu.VMEM((CH, D), jnp.float32),
        pltpu.VMEM_SHARED((N_NODES, D), jnp.float32),
        pltpu.SemaphoreType.DMA,
        pltpu.SemaphoreType.DMA,
        pltpu.SemaphoreType.DMA,
        pltpu.SemaphoreType.DMA,
        pltpu.SemaphoreType.DMA,
        pltpu.SemaphoreType.DMA,
        pltpu.SemaphoreType.DMA,
        pltpu.SemaphoreType.DMA,
        pltpu.SemaphoreType.DMA,
    ],
)
def _accum(x_hbm, src_hbm, dst_hbm, w_hbm, part_hbm,
           srcb, dstb, wb, rfa, rfb, rfc, rfd, acc_sh,
           bsem, g0, g1, g2, g3, s0, s1, s2, s3):
    cid = lax.axis_index("c")
    sid = lax.axis_index("s")
    wid = cid * NS + sid
    bufs = ((rfa, g0, s0), (rfb, g1, s1), (rfc, g2, s2), (rfd, g3, s3))

    def _zero_row(i, carry):
        for f in range(D // L):
            rfa[i, pl.ds(f * L, L)] = jnp.zeros((L,), jnp.float32)
        return carry

    lax.fori_loop(0, ZCH, _zero_row, 0)
    for k in range(NZCH // NS + 1):
        c = sid + NS * k

        @pl.when(c < NZCH)
        def _():
            pltpu.sync_copy(rfa.at[pl.ds(0, ZCH)],
                            acc_sh.at[pl.ds(c * ZCH, ZCH)])

    plsc.subcore_barrier()

    def _fire_block(blk, slot):
        off = wid * E_TILE + blk * E_BLK
        pltpu.async_copy(src_hbm.at[pl.ds(off, E_BLK)], srcb.at[slot], bsem)
        pltpu.async_copy(dst_hbm.at[wid * NBLK + blk], dstb.at[slot], bsem)
        pltpu.async_copy(w_hbm.at[pl.ds(off, E_BLK)], wb.at[slot], bsem)

    def _wait_block(blk, slot):
        off = wid * E_TILE + blk * E_BLK
        pltpu.make_async_copy(src_hbm.at[pl.ds(off, E_BLK)],
                              srcb.at[slot], bsem).wait()
        pltpu.make_async_copy(dst_hbm.at[wid * NBLK + blk],
                              dstb.at[slot], bsem).wait()
        pltpu.make_async_copy(w_hbm.at[pl.ds(off, E_BLK)],
                              wb.at[slot], bsem).wait()

    def _gather_src(slot, c):
        return x_hbm.at[srcb.at[slot, pl.ds(c * CH, CH)]]

    def _scale_chunk(slot, c, rows):
        def _cs(g, carry):
            wvec = wb[slot, pl.ds(c * CH + g * L, L)]
            for e16 in range(L):
                w = wvec[e16]
                e = g * L + e16
                for f in range(D // L):
                    sl = pl.ds(f * L, L)
                    rows[e, sl] = rows[e, sl] * w
            return carry

        lax.fori_loop(0, CH // L, _cs, 0)

    _fire_block(0, 0)
    _wait_block(0, 0)
    for b, (rows, gsem, ssem) in enumerate(bufs):
        pltpu.async_copy(_gather_src(0, b), rows, gsem)

    NB = len(bufs)

    def _quad(p, carry):
        j0 = NB * p
        blk = j0 // BLK
        slot = lax.rem(blk, 2)
        c0 = j0 - blk * BLK

        @pl.when((c0 == BLK - NB) & (blk + 1 < NBLK))
        def _():
            _wait_block(blk + 1, 1 - slot)

        for b, (rows, gsem, ssem) in enumerate(bufs):
            c = c0 + b
            pltpu.make_async_copy(_gather_src(slot, c), rows, gsem).wait()
            _scale_chunk(slot, c, rows)
            pltpu.async_copy(rows, acc_sh.at[dstb.at[slot, c]], ssem,
                             add=True)

        jn = j0 + NB
        blkn = jn // BLK
        slotn = lax.rem(blkn, 2)
        cn = jn - blkn * BLK

        for b, (rows, gsem, ssem) in enumerate(bufs):
            pltpu.make_async_copy(rows, acc_sh.at[dstb.at[slot, c0 + b]],
                                  ssem).wait()

            @pl.when(jn < NCHUNK)
            def _():
                pltpu.async_copy(_gather_src(slotn, cn + b), rows, gsem)

        @pl.when((c0 == 0) & (blk + 1 < NBLK))
        def _():
            _fire_block(blk + 1, 1 - slot)

        return carry

    lax.fori_loop(0, NCHUNK // NB, _quad, 0)

    plsc.subcore_barrier()

    for k in range(NZCH // NS + 1):
        c = sid + NS * k

        @pl.when(c < NZCH)
        def _():
            off = c * ZCH
            pltpu.sync_copy(acc_sh.at[pl.ds(off, ZCH)],
                            rfa.at[pl.ds(0, ZCH)])
            pltpu.sync_copy(rfa.at[pl.ds(0, ZCH)],
                            part_hbm.at[cid, pl.ds(off, ZCH)])


@functools.partial(
    pl.kernel,
    out_type=jax.ShapeDtypeStruct((N_NODES, D), jnp.float32),
    mesh=_mesh,
    scratch_types=[
        pltpu.VMEM((ZCH, D), jnp.float32),
        pltpu.VMEM((ZCH, D), jnp.float32),
        pltpu.VMEM((ZCH, D), jnp.float32),
    ],
)
def _combine(part_hbm, out_hbm, a_v, b_v, c_v):
    cid = lax.axis_index("c")
    sid = lax.axis_index("s")
    wid = cid * NS + sid

    for k in range(NZCH // NW + 1):
        c = wid + NW * k

        @pl.when(c < NZCH)
        def _():
            off = c * ZCH
            pltpu.sync_copy(part_hbm.at[0, pl.ds(off, ZCH)], a_v)
            pltpu.sync_copy(part_hbm.at[1, pl.ds(off, ZCH)], b_v)

            def _add(i, carry):
                for q in range(D // 32):
                    slo = pl.ds(32 * q, L)
                    shi = pl.ds(32 * q + 16, L)
                    c_v[i, slo] = a_v[i, slo] + b_v[i, slo]
                    c_v[i, shi] = a_v[i, shi] + b_v[i, shi]
                return carry

            lax.fori_loop(0, ZCH, _add, 0)
            pltpu.sync_copy(c_v, out_hbm.at[pl.ds(off, ZCH)])


def kernel(x, edge_index, edge_weight):
    dst = edge_index[0]
    src = edge_index[1]
    pad = E_PAD - N_EDGES
    pad_idx = jnp.arange(pad, dtype=jnp.int32) % N_NODES
    src_p = jnp.concatenate([src.astype(jnp.int32), pad_idx])
    dst_p = jnp.concatenate([dst.astype(jnp.int32), pad_idx]).reshape(
        NW * NBLK, BLK, CH)
    w_p = jnp.concatenate([edge_weight, jnp.zeros((pad,), jnp.float32)])
    part = _accum(x, src_p, dst_p, w_p)
    return _combine(part)
